# baseline (speedup 1.0000x reference)
import functools

import jax
import jax.numpy as jnp
from jax import lax
from jax.experimental import pallas as pl
from jax.experimental.pallas import tpu as pltpu
from jax.experimental.pallas import tpu_sc as plsc


def _make_pool_kernel(B, CTX, D):
    info = plsc.get_sparse_core_info()
    nw = info.num_cores * info.num_subcores
    rw = B // nw
    pw = rw * CTX
    ch = 80
    assert pw % ch == 0 and B % nw == 0
    nch = pw // ch

    mesh = plsc.VectorSubcoreMesh(core_axis_name="c", subcore_axis_name="s")

    @functools.partial(
        pl.kernel,
        out_type=jax.ShapeDtypeStruct((B, D), jnp.float32),
        mesh=mesh,
        compiler_params=pltpu.CompilerParams(use_tc_tiling_on_sc=False),
        scratch_types=[
            pltpu.VMEM((pw,), jnp.int32),
            pltpu.VMEM((pw, D), jnp.float32),
            pltpu.VMEM((rw, D), jnp.float32),
            pltpu.SemaphoreType.DMA,
        ],
    )
    def pool(cw_hbm, table_hbm, avg_hbm, idx_v, rows_v, out_v, sem):
        wid = lax.axis_index("s") * info.num_cores + lax.axis_index("c")
        pltpu.sync_copy(cw_hbm.at[pl.ds(wid * pw, pw)], idx_v)
        copies = [
            pltpu.async_copy(
                table_hbm.at[idx_v.at[pl.ds(c * ch, ch)]],
                rows_v.at[pl.ds(c * ch, ch), :],
                sem,
            )
            for c in range(nch)
        ]
        for cp in copies:
            cp.wait()

        scale = jnp.float32(1.0 / CTX)

        def row_body(r, carry):
            base = r * CTX
            acc = rows_v[base, :]
            for j in range(1, CTX):
                acc = acc + rows_v[base + j, :]
            out_v[r, :] = acc * scale
            return carry

        lax.fori_loop(0, rw, row_body, jnp.int32(0))
        pltpu.sync_copy(out_v, avg_hbm.at[pl.ds(wid * rw, rw), :])

    return pool


def _projection_t(avg, Wt, b2):
    B, D = avg.shape
    V = Wt.shape[1]
    vblk = 3072
    grid = pl.cdiv(V, vblk)

    def body(avg_ref, w_ref, b_ref, out_ref):
        out_ref[:, :] = (
            lax.dot_general(
                w_ref[:, :],
                avg_ref[:, :],
                (((0,), (1,)), ((), ())),
                preferred_element_type=jnp.float32,
            )
            + b_ref[0, :][:, None]
        )

    return pl.pallas_call(
        body,
        grid=(grid,),
        in_specs=[
            pl.BlockSpec((B, D), lambda i: (0, 0)),
            pl.BlockSpec((D, vblk), lambda i: (0, i)),
            pl.BlockSpec((1, vblk), lambda i: (0, i)),
        ],
        out_specs=pl.BlockSpec((vblk, B), lambda i: (i, 0)),
        out_shape=jax.ShapeDtypeStruct((V, B), jnp.float32),
        compiler_params=pltpu.CompilerParams(vmem_limit_bytes=100 * 1024 * 1024),
    )(avg, Wt, b2)


def kernel(context_words, emb_table, W, b):
    B, CTX = context_words.shape
    V, D = W.shape
    cw_flat = context_words.reshape(-1).astype(jnp.int32)
    pool = _make_pool_kernel(B, CTX, D)
    avg = pool(cw_flat, emb_table)
    out_t = _projection_t(avg, W.T, b.reshape(1, V))
    return out_t.T


# speedup vs baseline: 1.0034x; 1.0034x over previous
import functools

import jax
import jax.numpy as jnp
from jax import lax
from jax.experimental import pallas as pl
from jax.experimental.pallas import tpu as pltpu
from jax.experimental.pallas import tpu_sc as plsc


def _make_pool_kernel(B, CTX, D):
    info = plsc.get_sparse_core_info()
    nw = info.num_cores * info.num_subcores
    rw = B // nw
    pw = rw * CTX
    ch = 80
    assert pw % ch == 0 and B % nw == 0
    nch = pw // ch

    mesh = plsc.VectorSubcoreMesh(core_axis_name="c", subcore_axis_name="s")

    @functools.partial(
        pl.kernel,
        out_type=jax.ShapeDtypeStruct((B, D), jnp.float32),
        mesh=mesh,
        compiler_params=pltpu.CompilerParams(use_tc_tiling_on_sc=False),
        scratch_types=[
            pltpu.VMEM((pw,), jnp.int32),
            pltpu.VMEM((pw, D), jnp.float32),
            pltpu.VMEM((rw, D), jnp.float32),
            pltpu.SemaphoreType.DMA,
        ],
    )
    def pool(cw_hbm, table_hbm, avg_hbm, idx_v, rows_v, out_v, sem):
        wid = lax.axis_index("s") * info.num_cores + lax.axis_index("c")
        pltpu.sync_copy(cw_hbm.at[pl.ds(wid * pw, pw)], idx_v)
        copies = [
            pltpu.async_copy(
                table_hbm.at[idx_v.at[pl.ds(c * ch, ch)]],
                rows_v.at[pl.ds(c * ch, ch), :],
                sem,
            )
            for c in range(nch)
        ]
        for cp in copies:
            cp.wait()

        scale = jnp.float32(1.0 / CTX)

        def row_body(r, carry):
            base = r * CTX
            acc = rows_v[base, :]
            for j in range(1, CTX):
                acc = acc + rows_v[base + j, :]
            out_v[r, :] = acc * scale
            return carry

        lax.fori_loop(0, rw, row_body, jnp.int32(0))
        pltpu.sync_copy(out_v, avg_hbm.at[pl.ds(wid * rw, rw), :])

    return pool


def _projection_t(avg, Wt, b2):
    B, D = avg.shape
    V = Wt.shape[1]
    vblk = 2048
    grid = pl.cdiv(V, vblk)

    def body(avg_ref, w_ref, b_ref, out_ref):
        out_ref[:, :] = (
            lax.dot_general(
                w_ref[:, :],
                avg_ref[:, :],
                (((0,), (1,)), ((), ())),
                preferred_element_type=jnp.float32,
            )
            + b_ref[0, :][:, None]
        )

    return pl.pallas_call(
        body,
        grid=(grid,),
        in_specs=[
            pl.BlockSpec((B, D), lambda i: (0, 0)),
            pl.BlockSpec((D, vblk), lambda i: (0, i)),
            pl.BlockSpec((1, vblk), lambda i: (0, i)),
        ],
        out_specs=pl.BlockSpec((vblk, B), lambda i: (i, 0)),
        out_shape=jax.ShapeDtypeStruct((V, B), jnp.float32),
        compiler_params=pltpu.CompilerParams(vmem_limit_bytes=100 * 1024 * 1024),
    )(avg, Wt, b2)


def kernel(context_words, emb_table, W, b):
    B, CTX = context_words.shape
    V, D = W.shape
    cw_flat = context_words.reshape(-1).astype(jnp.int32)
    pool = _make_pool_kernel(B, CTX, D)
    avg = pool(cw_flat, emb_table)
    out_t = _projection_t(avg, W.T, b.reshape(1, V))
    return out_t.T
